# baseline (speedup 1.0000x reference)
import functools

import jax
import jax.numpy as jnp
from jax import lax
from jax.experimental import pallas as pl
from jax.experimental.pallas import tpu as pltpu
from jax.experimental.pallas import tpu_sc as plsc

N_NODES = 10000
N_EDGES = 320000
NPAD = 10240
ROWS_PER_TILE = NPAD // 16
EDGES_PER_TILE = N_EDGES // 16
BLK = 100
NBLK = EDGES_PER_TILE // BLK
NBLK_H = NBLK // 2
T_BLOCKS = (104, 96)
T_CH = 8
IN_CH = 128
HID_CH = 256
OUT_CH = 128
HALF = HID_CH // 2

_mesh = plsc.VectorSubcoreMesh(core_axis_name="c", subcore_axis_name="s")


def _zero_1d(ref, n):
    def body(k, _):
        ref[pl.ds(k * 16, 16)] = jnp.zeros((16,), jnp.float32)
        return 0
    lax.fori_loop(0, n // 16, body, 0)


def _fill_1d(ref, n, val):
    def body(k, _):
        ref[pl.ds(k * 16, 16)] = jnp.full((16,), val, jnp.float32)
        return 0
    lax.fori_loop(0, n // 16, body, 0)


def _zero_rows(ref, nrows):
    def body(r, _):
        for u in range(HALF // 16):
            ref[r, pl.ds(u * 16, 16)] = jnp.zeros((16,), jnp.float32)
        return 0
    lax.fori_loop(0, nrows, body, 0)


def _sc_deg_kernel(dst3d, deg0_out, deg1_out, idx_v, ones_v, z_v, deg_sp,
                   sem):
    c = lax.axis_index("c")
    s = lax.axis_index("s")

    _zero_1d(z_v, ROWS_PER_TILE)
    _fill_1d(ones_v, 128, 1.0)
    pltpu.sync_copy(z_v, deg_sp.at[pl.ds(s * ROWS_PER_TILE, ROWS_PER_TILE)])
    plsc.subcore_barrier()

    pltpu.sync_copy(dst3d.at[s], idx_v)

    GRP = 10

    def body(jg, _):
        row0 = c * NBLK_H + jg * GRP
        for u in range(GRP):
            pltpu.async_copy(ones_v.at[pl.ds(0, BLK)],
                             deg_sp.at[idx_v.at[row0 + u]], sem, add=True)
        for u in range(GRP):
            pltpu.make_async_copy(ones_v.at[pl.ds(0, BLK)],
                                  deg_sp.at[idx_v.at[row0 + u]], sem).wait()
        return 0
    lax.fori_loop(0, NBLK_H // GRP, body, 0)
    plsc.subcore_barrier()

    sl = pl.ds(s * ROWS_PER_TILE, ROWS_PER_TILE)

    @pl.when(c == 0)
    def _():
        pltpu.sync_copy(deg_sp.at[sl], deg0_out.at[sl])

    @pl.when(c == 1)
    def _():
        pltpu.sync_copy(deg_sp.at[sl], deg1_out.at[sl])


@functools.partial(
    pl.kernel,
    mesh=_mesh,
    out_type=[
        jax.ShapeDtypeStruct((NPAD,), jnp.float32),
        jax.ShapeDtypeStruct((NPAD,), jnp.float32),
    ],
    scratch_types=[
        pltpu.VMEM((NBLK, BLK), jnp.int32),
        pltpu.VMEM((128,), jnp.float32),
        pltpu.VMEM((ROWS_PER_TILE,), jnp.float32),
        pltpu.VMEM_SHARED((NPAD,), jnp.float32),
        pltpu.SemaphoreType.DMA,
    ],
)
def _sc_deg(dst3d, deg0_out, deg1_out, idx_v, ones_v, z_v, deg_sp, sem):
    _sc_deg_kernel(dst3d, deg0_out, deg1_out, idx_v, ones_v, z_v, deg_sp, sem)


def _tc_matmul_kernel(x_ref, w1_ref, xw0_ref, xw1_ref):
    xw = jnp.dot(x_ref[...], w1_ref[...], preferred_element_type=jnp.float32)
    xw0_ref[...] = xw[:, :HALF]
    xw1_ref[...] = xw[:, HALF:]


def _tc_matmul(x_pad, W1):
    R = 512
    grid = (NPAD // R,)
    return pl.pallas_call(
        _tc_matmul_kernel,
        grid=grid,
        in_specs=[
            pl.BlockSpec((R, IN_CH), lambda i: (i, 0)),
            pl.BlockSpec((IN_CH, HID_CH), lambda i: (0, 0)),
        ],
        out_specs=[
            pl.BlockSpec((R, HALF), lambda i: (i, 0)),
            pl.BlockSpec((R, HALF), lambda i: (i, 0)),
        ],
        out_shape=[
            jax.ShapeDtypeStruct((NPAD, HALF), jnp.float32),
            jax.ShapeDtypeStruct((NPAD, HALF), jnp.float32),
        ],
    )(x_pad, W1)


def _tc_scale_kernel(xw0_ref, xw1_ref, d0_ref, d1_ref,
                     dis_ref, xws0_ref, xws1_ref):
    deg = d0_ref[...] + d1_ref[...] + 1.0
    dis = lax.rsqrt(deg)
    dis_ref[...] = dis
    dis2 = jnp.reshape(dis, (dis.shape[0], 1))
    xws0_ref[...] = xw0_ref[...] * dis2
    xws1_ref[...] = xw1_ref[...] * dis2


def _tc_scale(xw0, xw1, deg0, deg1):
    R = 512
    grid = (NPAD // R,)
    return pl.pallas_call(
        _tc_scale_kernel,
        grid=grid,
        in_specs=[
            pl.BlockSpec((R, HALF), lambda i: (i, 0)),
            pl.BlockSpec((R, HALF), lambda i: (i, 0)),
            pl.BlockSpec((R,), lambda i: (i,)),
            pl.BlockSpec((R,), lambda i: (i,)),
        ],
        out_specs=[
            pl.BlockSpec((R,), lambda i: (i,)),
            pl.BlockSpec((R, HALF), lambda i: (i, 0)),
            pl.BlockSpec((R, HALF), lambda i: (i, 0)),
        ],
        out_shape=[
            jax.ShapeDtypeStruct((NPAD,), jnp.float32),
            jax.ShapeDtypeStruct((NPAD, HALF), jnp.float32),
            jax.ShapeDtypeStruct((NPAD, HALF), jnp.float32),
        ],
    )(xw0, xw1, deg0, deg1)


IDXCH = 8


def _k4_main_loop(xws_hbm, src3d, dst3d, s, src_c, dst_c,
                  bufs, acc_sp, sem_i, sem_g, sem_s):
    nch = NBLK // IDXCH

    ca = pltpu.async_copy(src3d.at[s, pl.ds(0, IDXCH)], src_c.at[0], sem_i)
    cb = pltpu.async_copy(dst3d.at[s, pl.ds(0, IDXCH)], dst_c.at[0], sem_i)
    ca.wait()
    cb.wait()
    pltpu.async_copy(src3d.at[s, pl.ds(IDXCH, IDXCH)], src_c.at[1], sem_i)
    pltpu.async_copy(dst3d.at[s, pl.ds(IDXCH, IDXCH)], dst_c.at[1], sem_i)
    pltpu.async_copy(xws_hbm.at[src_c.at[0, 0]], bufs[0], sem_g)
    pltpu.async_copy(xws_hbm.at[src_c.at[0, 1]], bufs[1], sem_g)

    def body(j, _):
        k = lax.div(j, IDXCH)
        r8 = lax.rem(j, IDXCH)
        p = lax.rem(k, 2)

        @pl.when(j > 0)
        def _():
            pltpu.make_async_copy(bufs[0], acc_sp.at[dst_c.at[0, 0]],
                                  sem_s).wait()

        @pl.when(jnp.logical_and(r8 == 0, jnp.logical_and(j > 0,
                                                          k < nch - 1)))
        def _():
            pltpu.async_copy(src3d.at[s, pl.ds((k + 1) * IDXCH, IDXCH)],
                             src_c.at[1 - p], sem_i)
            pltpu.async_copy(dst3d.at[s, pl.ds((k + 1) * IDXCH, IDXCH)],
                             dst_c.at[1 - p], sem_i)

        @pl.when(jnp.logical_and(r8 == IDXCH - 2, k < nch - 1))
        def _():
            pltpu.make_async_copy(src3d.at[s, pl.ds(0, IDXCH)],
                                  src_c.at[0], sem_i).wait()
            pltpu.make_async_copy(dst3d.at[s, pl.ds(0, IDXCH)],
                                  dst_c.at[0], sem_i).wait()

        def step(kk):
            buf = bufs[kk]
            pltpu.make_async_copy(xws_hbm.at[src_c.at[p, r8]],
                                  buf, sem_g).wait()
            pltpu.async_copy(buf, acc_sp.at[dst_c.at[p, r8]], sem_s,
                             add=True)

            @pl.when(j < NBLK - 2)
            def _():
                p2 = lax.rem(lax.div(j + 2, IDXCH), 2)
                r2 = lax.rem(j + 2, IDXCH)
                pltpu.async_copy(xws_hbm.at[src_c.at[p2, r2]],
                                 bufs[(kk + 2) % 3], sem_g)

        r3 = lax.rem(j, 3)

        @pl.when(r3 == 0)
        def _():
            step(0)

        @pl.when(r3 == 1)
        def _():
            step(1)

        @pl.when(r3 == 2)
        def _():
            step(2)
        return 0
    lax.fori_loop(0, NBLK, body, 0)
    pltpu.make_async_copy(bufs[0], acc_sp.at[dst_c.at[0, 0]], sem_s).wait()


def _k4_readout(acc_sp, t_sp, acc_out, t_out, s):
    rsl = pl.ds(s * ROWS_PER_TILE, ROWS_PER_TILE)
    pltpu.sync_copy(acc_sp.at[rsl], acc_out.at[rsl])
    pltpu.sync_copy(t_sp.at[rsl], t_out.at[rsl])


def _sc_scatter_kernel(src3d, dst3d, dis_hbm, xws0, xws1,
                       acc0_out, acc1_out, t0_out, t1_out,
                       src_c, dst_c, rowbufA, rowbufB, rowbufC,
                       valbufA, valbufB, z_v, acc_sp, t_sp, dis_sp,
                       sem_i, sem_g, sem_s):
    c = lax.axis_index("c")
    s = lax.axis_index("s")
    bufs = (rowbufA, rowbufB, rowbufC)

    _zero_rows(rowbufA, BLK)
    _zero_1d(z_v, ROWS_PER_TILE)
    base = s * ROWS_PER_TILE
    for k in range(ROWS_PER_TILE // 80):
        pltpu.sync_copy(rowbufA.at[pl.ds(0, 80)],
                        acc_sp.at[pl.ds(base + k * 80, 80)])
    pltpu.sync_copy(z_v, t_sp.at[pl.ds(base, ROWS_PER_TILE)])
    pltpu.sync_copy(dis_hbm.at[pl.ds(base, ROWS_PER_TILE)],
                    dis_sp.at[pl.ds(base, ROWS_PER_TILE)])
    plsc.subcore_barrier()

    vbufs = (valbufA, valbufB)
    base_t = c * T_BLOCKS[0]
    trips = jnp.where(c == 0, T_BLOCKS[0] // T_CH, T_BLOCKS[1] // T_CH)

    ca = pltpu.async_copy(src3d.at[s, pl.ds(base_t, T_CH)],
                          src_c.at[0], sem_i)
    cb = pltpu.async_copy(dst3d.at[s, pl.ds(base_t, T_CH)],
                          dst_c.at[0], sem_i)
    ca.wait()
    cb.wait()

    def t_outer(g, _):
        p = lax.rem(g, 2)

        @pl.when(g + 1 < trips)
        def _():
            row1 = base_t + (g + 1) * T_CH
            pltpu.async_copy(src3d.at[s, pl.ds(row1, T_CH)],
                             src_c.at[1 - p], sem_i)
            pltpu.async_copy(dst3d.at[s, pl.ds(row1, T_CH)],
                             dst_c.at[1 - p], sem_i)

        pend = pltpu.async_copy(dis_sp.at[dst_c.at[p, 0]],
                                vbufs[0].at[pl.ds(0, BLK)], sem_g)
        scat = []
        for j in range(T_CH):
            pend.wait()
            sc = pltpu.async_copy(vbufs[j % 2].at[pl.ds(0, BLK)],
                                  t_sp.at[src_c.at[p, j]], sem_s, add=True)
            scat.append(sc)
            if j + 1 < T_CH:
                if j >= 1:
                    scat[j - 1].wait()
                pend = pltpu.async_copy(
                    dis_sp.at[dst_c.at[p, j + 1]],
                    vbufs[(j + 1) % 2].at[pl.ds(0, BLK)], sem_g)
        scat[T_CH - 2].wait()
        scat[T_CH - 1].wait()

        @pl.when(g + 1 < trips)
        def _():
            pltpu.make_async_copy(src3d.at[s, pl.ds(base_t, T_CH)],
                                  src_c.at[0], sem_i).wait()
            pltpu.make_async_copy(dst3d.at[s, pl.ds(base_t, T_CH)],
                                  dst_c.at[0], sem_i).wait()
        return 0
    lax.fori_loop(0, trips, t_outer, 0)

    @pl.when(c == 0)
    def _():
        _k4_main_loop(xws0, src3d, dst3d, s, src_c, dst_c,
                      bufs, acc_sp, sem_i, sem_g, sem_s)

    @pl.when(c == 1)
    def _():
        _k4_main_loop(xws1, src3d, dst3d, s, src_c, dst_c,
                      bufs, acc_sp, sem_i, sem_g, sem_s)

    plsc.subcore_barrier()

    @pl.when(c == 0)
    def _():
        _k4_readout(acc_sp, t_sp, acc0_out, t0_out, s)

    @pl.when(c == 1)
    def _():
        _k4_readout(acc_sp, t_sp, acc1_out, t1_out, s)


@functools.partial(
    pl.kernel,
    mesh=_mesh,
    out_type=[
        jax.ShapeDtypeStruct((NPAD, HALF), jnp.float32),
        jax.ShapeDtypeStruct((NPAD, HALF), jnp.float32),
        jax.ShapeDtypeStruct((NPAD,), jnp.float32),
        jax.ShapeDtypeStruct((NPAD,), jnp.float32),
    ],
    scratch_types=[
        pltpu.VMEM((2, IDXCH, BLK), jnp.int32),
        pltpu.VMEM((2, IDXCH, BLK), jnp.int32),
        pltpu.VMEM((BLK, HALF), jnp.float32),
        pltpu.VMEM((BLK, HALF), jnp.float32),
        pltpu.VMEM((BLK, HALF), jnp.float32),
        pltpu.VMEM((128,), jnp.float32),
        pltpu.VMEM((128,), jnp.float32),
        pltpu.VMEM((ROWS_PER_TILE,), jnp.float32),
        pltpu.VMEM_SHARED((NPAD, HALF), jnp.float32),
        pltpu.VMEM_SHARED((NPAD,), jnp.float32),
        pltpu.VMEM_SHARED((NPAD,), jnp.float32),
        pltpu.SemaphoreType.DMA,
        pltpu.SemaphoreType.DMA,
        pltpu.SemaphoreType.DMA,
    ],
)
def _sc_scatter(src3d, dst3d, dis_hbm, xws0, xws1,
                acc0_out, acc1_out, t0_out, t1_out,
                src_c, dst_c, rowbufA, rowbufB, rowbufC,
                valbufA, valbufB, z_v, acc_sp, t_sp, dis_sp,
                sem_i, sem_g, sem_s):
    _sc_scatter_kernel(src3d, dst3d, dis_hbm, xws0, xws1,
                       acc0_out, acc1_out, t0_out, t1_out,
                       src_c, dst_c, rowbufA, rowbufB, rowbufC,
                       valbufA, valbufB, z_v, acc_sp, t_sp, dis_sp,
                       sem_i, sem_g, sem_s)


def _tc_final_kernel(acc0_ref, acc1_ref, xws0_ref, xws1_ref, dis_ref,
                     t0_ref, t1_ref, b1_ref, w2_ref, b2_ref,
                     out_ref, vacc):
    i = pl.program_id(0)
    R = acc0_ref.shape[0]
    dis = dis_ref[...]
    dis2 = jnp.reshape(dis, (R, 1))
    b1 = jnp.reshape(b1_ref[...], (1, HID_CH))
    h0 = jnp.maximum(dis2 * (acc0_ref[...] + xws0_ref[...]) + b1[:, :HALF],
                     0.0)
    h1 = jnp.maximum(dis2 * (acc1_ref[...] + xws1_ref[...]) + b1[:, HALF:],
                     0.0)
    cvec = dis * (t0_ref[...] + t1_ref[...] + dis)
    cvec2 = jnp.reshape(cvec, (R, 1))
    rows = lax.broadcasted_iota(jnp.int32, (R, 1), 0) + i * R
    cvec2 = jnp.where(rows < N_NODES, cvec2, 0.0)
    contrib = jnp.concatenate(
        [jnp.sum(cvec2 * h0, axis=0, keepdims=True),
         jnp.sum(cvec2 * h1, axis=0, keepdims=True)], axis=1)

    @pl.when(i == 0)
    def _():
        vacc[...] = contrib

    @pl.when(i > 0)
    def _():
        vacc[...] = vacc[...] + contrib

    @pl.when(i == pl.num_programs(0) - 1)
    def _():
        v = vacc[...]
        o = jnp.dot(v, w2_ref[...], preferred_element_type=jnp.float32)
        out_ref[...] = o * (1.0 / N_NODES) + b2_ref[...]


def _tc_final(acc0, acc1, xws0, xws1, dis, t0, t1, b1, W2, b2):
    R = 512
    grid = (NPAD // R,)
    return pl.pallas_call(
        _tc_final_kernel,
        grid=grid,
        in_specs=[
            pl.BlockSpec((R, HALF), lambda i: (i, 0)),
            pl.BlockSpec((R, HALF), lambda i: (i, 0)),
            pl.BlockSpec((R, HALF), lambda i: (i, 0)),
            pl.BlockSpec((R, HALF), lambda i: (i, 0)),
            pl.BlockSpec((R,), lambda i: (i,)),
            pl.BlockSpec((R,), lambda i: (i,)),
            pl.BlockSpec((R,), lambda i: (i,)),
            pl.BlockSpec((HID_CH,), lambda i: (0,)),
            pl.BlockSpec((HID_CH, OUT_CH), lambda i: (0, 0)),
            pl.BlockSpec((1, OUT_CH), lambda i: (0, 0)),
        ],
        out_specs=pl.BlockSpec((1, OUT_CH), lambda i: (0, 0)),
        out_shape=jax.ShapeDtypeStruct((1, OUT_CH), jnp.float32),
        scratch_shapes=[pltpu.VMEM((1, HID_CH), jnp.float32)],
    )(acc0, acc1, xws0, xws1, dis, t0, t1, b1, W2, b2)


def kernel(x, edge_index, W1, b1, W2, b2):
    src = edge_index[0].astype(jnp.int32)
    dst = edge_index[1].astype(jnp.int32)
    src3d = src.reshape(16, NBLK, BLK)
    dst3d = dst.reshape(16, NBLK, BLK)
    x_pad = jnp.pad(x, ((0, NPAD - N_NODES), (0, 0)))

    xw0, xw1 = _tc_matmul(x_pad, W1)
    deg0, deg1 = _sc_deg(dst3d)
    dis, xws0, xws1 = _tc_scale(xw0, xw1, deg0, deg1)
    acc0, acc1, t0, t1 = _sc_scatter(src3d, dst3d, dis, xws0, xws1)
    out = _tc_final(acc0, acc1, xws0, xws1, dis, t0, t1, b1, W2,
                    b2.reshape(1, OUT_CH))
    return out.reshape(OUT_CH)


# speedup vs baseline: 1.0660x; 1.0660x over previous
import functools

import jax
import jax.numpy as jnp
from jax import lax
from jax.experimental import pallas as pl
from jax.experimental.pallas import tpu as pltpu
from jax.experimental.pallas import tpu_sc as plsc

N_NODES = 10000
N_EDGES = 320000
NPAD = 10240
ROWS_PER_TILE = NPAD // 16
EDGES_PER_TILE = N_EDGES // 16
BLK = 100
NBLK = EDGES_PER_TILE // BLK
NBLK_H = NBLK // 2
T_BLOCKS = (104, 96)
T_CH = 8
IN_CH = 128
HID_CH = 256
OUT_CH = 128
HALF = HID_CH // 2

_mesh = plsc.VectorSubcoreMesh(core_axis_name="c", subcore_axis_name="s")


def _zero_1d(ref, n):
    def body(k, _):
        ref[pl.ds(k * 16, 16)] = jnp.zeros((16,), jnp.float32)
        return 0
    lax.fori_loop(0, n // 16, body, 0)


def _fill_1d(ref, n, val):
    def body(k, _):
        ref[pl.ds(k * 16, 16)] = jnp.full((16,), val, jnp.float32)
        return 0
    lax.fori_loop(0, n // 16, body, 0)


def _zero_rows(ref, nrows):
    def body(r, _):
        for u in range(HALF // 16):
            ref[r, pl.ds(u * 16, 16)] = jnp.zeros((16,), jnp.float32)
        return 0
    lax.fori_loop(0, nrows, body, 0)


def _sc_deg_kernel(dst3d, deg0_out, deg1_out, idx_v, ones_v, z_v, deg_sp,
                   sem):
    c = lax.axis_index("c")
    s = lax.axis_index("s")

    _zero_1d(z_v, ROWS_PER_TILE)
    _fill_1d(ones_v, 128, 1.0)
    pltpu.sync_copy(z_v, deg_sp.at[pl.ds(s * ROWS_PER_TILE, ROWS_PER_TILE)])
    plsc.subcore_barrier()

    pltpu.sync_copy(dst3d.at[s], idx_v)

    GRP = 10

    def body(jg, _):
        row0 = c * NBLK_H + jg * GRP
        for u in range(GRP):
            pltpu.async_copy(ones_v.at[pl.ds(0, BLK)],
                             deg_sp.at[idx_v.at[row0 + u]], sem, add=True)
        for u in range(GRP):
            pltpu.make_async_copy(ones_v.at[pl.ds(0, BLK)],
                                  deg_sp.at[idx_v.at[row0 + u]], sem).wait()
        return 0
    lax.fori_loop(0, NBLK_H // GRP, body, 0)
    plsc.subcore_barrier()

    sl = pl.ds(s * ROWS_PER_TILE, ROWS_PER_TILE)

    @pl.when(c == 0)
    def _():
        pltpu.sync_copy(deg_sp.at[sl], deg0_out.at[sl])

    @pl.when(c == 1)
    def _():
        pltpu.sync_copy(deg_sp.at[sl], deg1_out.at[sl])


@functools.partial(
    pl.kernel,
    mesh=_mesh,
    out_type=[
        jax.ShapeDtypeStruct((NPAD,), jnp.float32),
        jax.ShapeDtypeStruct((NPAD,), jnp.float32),
    ],
    scratch_types=[
        pltpu.VMEM((NBLK, BLK), jnp.int32),
        pltpu.VMEM((128,), jnp.float32),
        pltpu.VMEM((ROWS_PER_TILE,), jnp.float32),
        pltpu.VMEM_SHARED((NPAD,), jnp.float32),
        pltpu.SemaphoreType.DMA,
    ],
)
def _sc_deg(dst3d, deg0_out, deg1_out, idx_v, ones_v, z_v, deg_sp, sem):
    _sc_deg_kernel(dst3d, deg0_out, deg1_out, idx_v, ones_v, z_v, deg_sp, sem)


def _tc_scale_kernel(x_ref, w1_ref, d0_ref, d1_ref,
                     dis_ref, xws0_ref, xws1_ref):
    xw = jnp.dot(x_ref[...], w1_ref[...], preferred_element_type=jnp.float32)
    deg = d0_ref[...] + d1_ref[...] + 1.0
    dis = lax.rsqrt(deg)
    dis_ref[...] = dis
    xws = xw * jnp.reshape(dis, (dis.shape[0], 1))
    xws0_ref[...] = xws[:, :HALF]
    xws1_ref[...] = xws[:, HALF:]


def _tc_scale(x_pad, W1, deg0, deg1):
    R = 512
    grid = (NPAD // R,)
    return pl.pallas_call(
        _tc_scale_kernel,
        grid=grid,
        in_specs=[
            pl.BlockSpec((R, IN_CH), lambda i: (i, 0)),
            pl.BlockSpec((IN_CH, HID_CH), lambda i: (0, 0)),
            pl.BlockSpec((R,), lambda i: (i,)),
            pl.BlockSpec((R,), lambda i: (i,)),
        ],
        out_specs=[
            pl.BlockSpec((R,), lambda i: (i,)),
            pl.BlockSpec((R, HALF), lambda i: (i, 0)),
            pl.BlockSpec((R, HALF), lambda i: (i, 0)),
        ],
        out_shape=[
            jax.ShapeDtypeStruct((NPAD,), jnp.float32),
            jax.ShapeDtypeStruct((NPAD, HALF), jnp.float32),
            jax.ShapeDtypeStruct((NPAD, HALF), jnp.float32),
        ],
    )(x_pad, W1, deg0, deg1)


IDXCH = 8


def _k4_main_loop(xws_hbm, src3d, dst3d, s, src_c, dst_c,
                  bufs, acc_sp, sem_i, sem_g, sem_s):
    nch = NBLK // IDXCH

    ca = pltpu.async_copy(src3d.at[s, pl.ds(0, IDXCH)], src_c.at[0], sem_i)
    cb = pltpu.async_copy(dst3d.at[s, pl.ds(0, IDXCH)], dst_c.at[0], sem_i)
    ca.wait()
    cb.wait()
    pltpu.async_copy(src3d.at[s, pl.ds(IDXCH, IDXCH)], src_c.at[1], sem_i)
    pltpu.async_copy(dst3d.at[s, pl.ds(IDXCH, IDXCH)], dst_c.at[1], sem_i)
    pltpu.async_copy(xws_hbm.at[src_c.at[0, 0]], bufs[0], sem_g)
    pltpu.async_copy(xws_hbm.at[src_c.at[0, 1]], bufs[1], sem_g)

    def body(j, _):
        k = lax.div(j, IDXCH)
        r8 = lax.rem(j, IDXCH)
        p = lax.rem(k, 2)

        @pl.when(j > 0)
        def _():
            pltpu.make_async_copy(bufs[0], acc_sp.at[dst_c.at[0, 0]],
                                  sem_s).wait()

        @pl.when(jnp.logical_and(r8 == 0, jnp.logical_and(j > 0,
                                                          k < nch - 1)))
        def _():
            pltpu.async_copy(src3d.at[s, pl.ds((k + 1) * IDXCH, IDXCH)],
                             src_c.at[1 - p], sem_i)
            pltpu.async_copy(dst3d.at[s, pl.ds((k + 1) * IDXCH, IDXCH)],
                             dst_c.at[1 - p], sem_i)

        @pl.when(jnp.logical_and(r8 == IDXCH - 2, k < nch - 1))
        def _():
            pltpu.make_async_copy(src3d.at[s, pl.ds(0, IDXCH)],
                                  src_c.at[0], sem_i).wait()
            pltpu.make_async_copy(dst3d.at[s, pl.ds(0, IDXCH)],
                                  dst_c.at[0], sem_i).wait()

        def step(kk):
            buf = bufs[kk]
            pltpu.make_async_copy(xws_hbm.at[src_c.at[p, r8]],
                                  buf, sem_g).wait()
            pltpu.async_copy(buf, acc_sp.at[dst_c.at[p, r8]], sem_s,
                             add=True)

            @pl.when(j < NBLK - 2)
            def _():
                p2 = lax.rem(lax.div(j + 2, IDXCH), 2)
                r2 = lax.rem(j + 2, IDXCH)
                pltpu.async_copy(xws_hbm.at[src_c.at[p2, r2]],
                                 bufs[(kk + 2) % 3], sem_g)

        r3 = lax.rem(j, 3)

        @pl.when(r3 == 0)
        def _():
            step(0)

        @pl.when(r3 == 1)
        def _():
            step(1)

        @pl.when(r3 == 2)
        def _():
            step(2)
        return 0
    lax.fori_loop(0, NBLK, body, 0)
    pltpu.make_async_copy(bufs[0], acc_sp.at[dst_c.at[0, 0]], sem_s).wait()


def _k4_readout(acc_sp, t_sp, acc_out, t_out, s):
    rsl = pl.ds(s * ROWS_PER_TILE, ROWS_PER_TILE)
    pltpu.sync_copy(acc_sp.at[rsl], acc_out.at[rsl])
    pltpu.sync_copy(t_sp.at[rsl], t_out.at[rsl])


def _sc_scatter_kernel(src3d, dst3d, dis_hbm, xws0, xws1,
                       acc0_out, acc1_out, t0_out, t1_out,
                       src_c, dst_c, rowbufA, rowbufB, rowbufC,
                       valbufA, valbufB, z_v, acc_sp, t_sp, dis_sp,
                       sem_i, sem_g, sem_s):
    c = lax.axis_index("c")
    s = lax.axis_index("s")
    bufs = (rowbufA, rowbufB, rowbufC)

    _zero_rows(rowbufA, BLK)
    _zero_1d(z_v, ROWS_PER_TILE)
    base = s * ROWS_PER_TILE
    for k in range(ROWS_PER_TILE // 80):
        pltpu.sync_copy(rowbufA.at[pl.ds(0, 80)],
                        acc_sp.at[pl.ds(base + k * 80, 80)])
    pltpu.sync_copy(z_v, t_sp.at[pl.ds(base, ROWS_PER_TILE)])
    pltpu.sync_copy(dis_hbm.at[pl.ds(base, ROWS_PER_TILE)],
                    dis_sp.at[pl.ds(base, ROWS_PER_TILE)])
    plsc.subcore_barrier()

    vbufs = (valbufA, valbufB)
    base_t = c * T_BLOCKS[0]
    trips = jnp.where(c == 0, T_BLOCKS[0] // T_CH, T_BLOCKS[1] // T_CH)

    ca = pltpu.async_copy(src3d.at[s, pl.ds(base_t, T_CH)],
                          src_c.at[0], sem_i)
    cb = pltpu.async_copy(dst3d.at[s, pl.ds(base_t, T_CH)],
                          dst_c.at[0], sem_i)
    ca.wait()
    cb.wait()

    def t_outer(g, _):
        p = lax.rem(g, 2)

        @pl.when(g + 1 < trips)
        def _():
            row1 = base_t + (g + 1) * T_CH
            pltpu.async_copy(src3d.at[s, pl.ds(row1, T_CH)],
                             src_c.at[1 - p], sem_i)
            pltpu.async_copy(dst3d.at[s, pl.ds(row1, T_CH)],
                             dst_c.at[1 - p], sem_i)

        pend = pltpu.async_copy(dis_sp.at[dst_c.at[p, 0]],
                                vbufs[0].at[pl.ds(0, BLK)], sem_g)
        scat = []
        for j in range(T_CH):
            pend.wait()
            sc = pltpu.async_copy(vbufs[j % 2].at[pl.ds(0, BLK)],
                                  t_sp.at[src_c.at[p, j]], sem_s, add=True)
            scat.append(sc)
            if j + 1 < T_CH:
                if j >= 1:
                    scat[j - 1].wait()
                pend = pltpu.async_copy(
                    dis_sp.at[dst_c.at[p, j + 1]],
                    vbufs[(j + 1) % 2].at[pl.ds(0, BLK)], sem_g)
        scat[T_CH - 2].wait()
        scat[T_CH - 1].wait()

        @pl.when(g + 1 < trips)
        def _():
            pltpu.make_async_copy(src3d.at[s, pl.ds(base_t, T_CH)],
                                  src_c.at[0], sem_i).wait()
            pltpu.make_async_copy(dst3d.at[s, pl.ds(base_t, T_CH)],
                                  dst_c.at[0], sem_i).wait()
        return 0
    lax.fori_loop(0, trips, t_outer, 0)

    @pl.when(c == 0)
    def _():
        _k4_main_loop(xws0, src3d, dst3d, s, src_c, dst_c,
                      bufs, acc_sp, sem_i, sem_g, sem_s)

    @pl.when(c == 1)
    def _():
        _k4_main_loop(xws1, src3d, dst3d, s, src_c, dst_c,
                      bufs, acc_sp, sem_i, sem_g, sem_s)

    plsc.subcore_barrier()

    @pl.when(c == 0)
    def _():
        _k4_readout(acc_sp, t_sp, acc0_out, t0_out, s)

    @pl.when(c == 1)
    def _():
        _k4_readout(acc_sp, t_sp, acc1_out, t1_out, s)


@functools.partial(
    pl.kernel,
    mesh=_mesh,
    out_type=[
        jax.ShapeDtypeStruct((NPAD, HALF), jnp.float32),
        jax.ShapeDtypeStruct((NPAD, HALF), jnp.float32),
        jax.ShapeDtypeStruct((NPAD,), jnp.float32),
        jax.ShapeDtypeStruct((NPAD,), jnp.float32),
    ],
    scratch_types=[
        pltpu.VMEM((2, IDXCH, BLK), jnp.int32),
        pltpu.VMEM((2, IDXCH, BLK), jnp.int32),
        pltpu.VMEM((BLK, HALF), jnp.float32),
        pltpu.VMEM((BLK, HALF), jnp.float32),
        pltpu.VMEM((BLK, HALF), jnp.float32),
        pltpu.VMEM((128,), jnp.float32),
        pltpu.VMEM((128,), jnp.float32),
        pltpu.VMEM((ROWS_PER_TILE,), jnp.float32),
        pltpu.VMEM_SHARED((NPAD, HALF), jnp.float32),
        pltpu.VMEM_SHARED((NPAD,), jnp.float32),
        pltpu.VMEM_SHARED((NPAD,), jnp.float32),
        pltpu.SemaphoreType.DMA,
        pltpu.SemaphoreType.DMA,
        pltpu.SemaphoreType.DMA,
    ],
)
def _sc_scatter(src3d, dst3d, dis_hbm, xws0, xws1,
                acc0_out, acc1_out, t0_out, t1_out,
                src_c, dst_c, rowbufA, rowbufB, rowbufC,
                valbufA, valbufB, z_v, acc_sp, t_sp, dis_sp,
                sem_i, sem_g, sem_s):
    _sc_scatter_kernel(src3d, dst3d, dis_hbm, xws0, xws1,
                       acc0_out, acc1_out, t0_out, t1_out,
                       src_c, dst_c, rowbufA, rowbufB, rowbufC,
                       valbufA, valbufB, z_v, acc_sp, t_sp, dis_sp,
                       sem_i, sem_g, sem_s)


def _tc_final_kernel(acc0_ref, acc1_ref, xws0_ref, xws1_ref, dis_ref,
                     t0_ref, t1_ref, b1_ref, w2_ref, b2_ref,
                     out_ref, vacc):
    i = pl.program_id(0)
    R = acc0_ref.shape[0]
    dis = dis_ref[...]
    dis2 = jnp.reshape(dis, (R, 1))
    b1 = jnp.reshape(b1_ref[...], (1, HID_CH))
    h0 = jnp.maximum(dis2 * (acc0_ref[...] + xws0_ref[...]) + b1[:, :HALF],
                     0.0)
    h1 = jnp.maximum(dis2 * (acc1_ref[...] + xws1_ref[...]) + b1[:, HALF:],
                     0.0)
    cvec = dis * (t0_ref[...] + t1_ref[...] + dis)
    cvec2 = jnp.reshape(cvec, (R, 1))
    rows = lax.broadcasted_iota(jnp.int32, (R, 1), 0) + i * R
    cvec2 = jnp.where(rows < N_NODES, cvec2, 0.0)
    contrib = jnp.concatenate(
        [jnp.sum(cvec2 * h0, axis=0, keepdims=True),
         jnp.sum(cvec2 * h1, axis=0, keepdims=True)], axis=1)

    @pl.when(i == 0)
    def _():
        vacc[...] = contrib

    @pl.when(i > 0)
    def _():
        vacc[...] = vacc[...] + contrib

    @pl.when(i == pl.num_programs(0) - 1)
    def _():
        v = vacc[...]
        o = jnp.dot(v, w2_ref[...], preferred_element_type=jnp.float32)
        out_ref[...] = o * (1.0 / N_NODES) + b2_ref[...]


def _tc_final(acc0, acc1, xws0, xws1, dis, t0, t1, b1, W2, b2):
    R = 512
    grid = (NPAD // R,)
    return pl.pallas_call(
        _tc_final_kernel,
        grid=grid,
        in_specs=[
            pl.BlockSpec((R, HALF), lambda i: (i, 0)),
            pl.BlockSpec((R, HALF), lambda i: (i, 0)),
            pl.BlockSpec((R, HALF), lambda i: (i, 0)),
            pl.BlockSpec((R, HALF), lambda i: (i, 0)),
            pl.BlockSpec((R,), lambda i: (i,)),
            pl.BlockSpec((R,), lambda i: (i,)),
            pl.BlockSpec((R,), lambda i: (i,)),
            pl.BlockSpec((HID_CH,), lambda i: (0,)),
            pl.BlockSpec((HID_CH, OUT_CH), lambda i: (0, 0)),
            pl.BlockSpec((1, OUT_CH), lambda i: (0, 0)),
        ],
        out_specs=pl.BlockSpec((1, OUT_CH), lambda i: (0, 0)),
        out_shape=jax.ShapeDtypeStruct((1, OUT_CH), jnp.float32),
        scratch_shapes=[pltpu.VMEM((1, HID_CH), jnp.float32)],
    )(acc0, acc1, xws0, xws1, dis, t0, t1, b1, W2, b2)


def kernel(x, edge_index, W1, b1, W2, b2):
    src = edge_index[0].astype(jnp.int32)
    dst = edge_index[1].astype(jnp.int32)
    src3d = src.reshape(16, NBLK, BLK)
    dst3d = dst.reshape(16, NBLK, BLK)
    x_pad = jnp.pad(x, ((0, NPAD - N_NODES), (0, 0)))

    deg0, deg1 = _sc_deg(dst3d)
    dis, xws0, xws1 = _tc_scale(x_pad, W1, deg0, deg1)
    acc0, acc1, t0, t1 = _sc_scatter(src3d, dst3d, dis, xws0, xws1)
    out = _tc_final(acc0, acc1, xws0, xws1, dis, t0, t1, b1, W2,
                    b2.reshape(1, OUT_CH))
    return out.reshape(OUT_CH)


# speedup vs baseline: 1.1288x; 1.0589x over previous
import functools

import jax
import jax.numpy as jnp
from jax import lax
from jax.experimental import pallas as pl
from jax.experimental.pallas import tpu as pltpu
from jax.experimental.pallas import tpu_sc as plsc

N_NODES = 10000
N_EDGES = 320000
NPAD = 10240
ROWS_PER_TILE = NPAD // 16
EDGES_PER_TILE = N_EDGES // 16
BLK = 100
NBLK = EDGES_PER_TILE // BLK
NBLK_H = NBLK // 2
T_BLOCKS = (104, 96)
T_CH = 8
IN_CH = 128
HID_CH = 256
OUT_CH = 128
HALF = HID_CH // 2

_mesh = plsc.VectorSubcoreMesh(core_axis_name="c", subcore_axis_name="s")


def _zero_1d(ref, n):
    def body(k, _):
        ref[pl.ds(k * 16, 16)] = jnp.zeros((16,), jnp.float32)
        return 0
    lax.fori_loop(0, n // 16, body, 0)


def _fill_1d(ref, n, val):
    def body(k, _):
        ref[pl.ds(k * 16, 16)] = jnp.full((16,), val, jnp.float32)
        return 0
    lax.fori_loop(0, n // 16, body, 0)


def _zero_rows(ref, nrows):
    def body(r, _):
        for u in range(HALF // 16):
            ref[r, pl.ds(u * 16, 16)] = jnp.zeros((16,), jnp.float32)
        return 0
    lax.fori_loop(0, nrows, body, 0)


def _sc_deg_kernel(dst3d, deg0_out, deg1_out, idx_v, ones_v, z_v, deg_sp,
                   sem):
    c = lax.axis_index("c")
    s = lax.axis_index("s")

    _zero_1d(z_v, ROWS_PER_TILE)
    _fill_1d(ones_v, 128, 1.0)
    pltpu.sync_copy(z_v, deg_sp.at[pl.ds(s * ROWS_PER_TILE, ROWS_PER_TILE)])
    plsc.subcore_barrier()

    pltpu.sync_copy(dst3d.at[s], idx_v)

    GRP = 10

    def body(jg, _):
        row0 = c * NBLK_H + jg * GRP
        for u in range(GRP):
            pltpu.async_copy(ones_v.at[pl.ds(0, BLK)],
                             deg_sp.at[idx_v.at[row0 + u]], sem, add=True)
        for u in range(GRP):
            pltpu.make_async_copy(ones_v.at[pl.ds(0, BLK)],
                                  deg_sp.at[idx_v.at[row0 + u]], sem).wait()
        return 0
    lax.fori_loop(0, NBLK_H // GRP, body, 0)
    plsc.subcore_barrier()

    sl = pl.ds(s * ROWS_PER_TILE, ROWS_PER_TILE)

    @pl.when(c == 0)
    def _():
        pltpu.sync_copy(deg_sp.at[sl], deg0_out.at[sl])

    @pl.when(c == 1)
    def _():
        pltpu.sync_copy(deg_sp.at[sl], deg1_out.at[sl])


@functools.partial(
    pl.kernel,
    mesh=_mesh,
    out_type=[
        jax.ShapeDtypeStruct((NPAD,), jnp.float32),
        jax.ShapeDtypeStruct((NPAD,), jnp.float32),
    ],
    scratch_types=[
        pltpu.VMEM((NBLK, BLK), jnp.int32),
        pltpu.VMEM((128,), jnp.float32),
        pltpu.VMEM((ROWS_PER_TILE,), jnp.float32),
        pltpu.VMEM_SHARED((NPAD,), jnp.float32),
        pltpu.SemaphoreType.DMA,
    ],
)
def _sc_deg(dst3d, deg0_out, deg1_out, idx_v, ones_v, z_v, deg_sp, sem):
    _sc_deg_kernel(dst3d, deg0_out, deg1_out, idx_v, ones_v, z_v, deg_sp, sem)


def _tc_scale_kernel(x_ref, w1_ref, d0_ref, d1_ref,
                     dis_ref, xws0_ref, xws1_ref):
    xw = jnp.dot(x_ref[...], w1_ref[...], preferred_element_type=jnp.float32)
    deg = d0_ref[...] + d1_ref[...] + 1.0
    dis = lax.rsqrt(deg)
    dis_ref[...] = dis
    xws = xw * jnp.reshape(dis, (dis.shape[0], 1))
    xws0_ref[...] = xws[:, :HALF]
    xws1_ref[...] = xws[:, HALF:]


def _tc_scale(x_pad, W1, deg0, deg1):
    R = 512
    grid = (NPAD // R,)
    return pl.pallas_call(
        _tc_scale_kernel,
        grid=grid,
        in_specs=[
            pl.BlockSpec((R, IN_CH), lambda i: (i, 0)),
            pl.BlockSpec((IN_CH, HID_CH), lambda i: (0, 0)),
            pl.BlockSpec((R,), lambda i: (i,)),
            pl.BlockSpec((R,), lambda i: (i,)),
        ],
        out_specs=[
            pl.BlockSpec((R,), lambda i: (i,)),
            pl.BlockSpec((R, HALF), lambda i: (i, 0)),
            pl.BlockSpec((R, HALF), lambda i: (i, 0)),
        ],
        out_shape=[
            jax.ShapeDtypeStruct((NPAD,), jnp.float32),
            jax.ShapeDtypeStruct((NPAD, HALF), jnp.float32),
            jax.ShapeDtypeStruct((NPAD, HALF), jnp.float32),
        ],
    )(x_pad, W1, deg0, deg1)


IDXCH = 8


def _k4_main_loop(xws_hbm, src3d, dst3d, s, c, src_c, dst_c,
                  bufs, vbuf, acc_sp, t_sp, dis_sp,
                  sem_i, sem_g, sem_s, sem_t, sem_ts):
    nch = NBLK // IDXCH
    t_start = c * T_BLOCKS[0]
    t_end = t_start + jnp.where(c == 0, T_BLOCKS[0], T_BLOCKS[1])

    ca = pltpu.async_copy(src3d.at[s, pl.ds(0, IDXCH)], src_c.at[0], sem_i)
    cb = pltpu.async_copy(dst3d.at[s, pl.ds(0, IDXCH)], dst_c.at[0], sem_i)
    ca.wait()
    cb.wait()
    pltpu.async_copy(src3d.at[s, pl.ds(IDXCH, IDXCH)], src_c.at[1], sem_i)
    pltpu.async_copy(dst3d.at[s, pl.ds(IDXCH, IDXCH)], dst_c.at[1], sem_i)
    pltpu.async_copy(xws_hbm.at[src_c.at[0, 0]], bufs[0], sem_g)
    pltpu.async_copy(xws_hbm.at[src_c.at[0, 1]], bufs[1], sem_g)

    def body(j, _):
        k = lax.div(j, IDXCH)
        r8 = lax.rem(j, IDXCH)
        p = lax.rem(k, 2)
        in_t = jnp.logical_and(j >= t_start, j < t_end)

        @pl.when(jnp.logical_and(j > t_start, j <= t_end))
        def _():
            pltpu.make_async_copy(vbuf.at[pl.ds(0, BLK)],
                                  t_sp.at[src_c.at[0, 0]], sem_ts).wait()

        @pl.when(in_t)
        def _():
            pltpu.async_copy(dis_sp.at[dst_c.at[p, r8]],
                             vbuf.at[pl.ds(0, BLK)], sem_t)

        @pl.when(j > 0)
        def _():
            pltpu.make_async_copy(bufs[0], acc_sp.at[dst_c.at[0, 0]],
                                  sem_s).wait()

        @pl.when(jnp.logical_and(r8 == 0, jnp.logical_and(j > 0,
                                                          k < nch - 1)))
        def _():
            pltpu.async_copy(src3d.at[s, pl.ds((k + 1) * IDXCH, IDXCH)],
                             src_c.at[1 - p], sem_i)
            pltpu.async_copy(dst3d.at[s, pl.ds((k + 1) * IDXCH, IDXCH)],
                             dst_c.at[1 - p], sem_i)

        @pl.when(jnp.logical_and(r8 == IDXCH - 2, k < nch - 1))
        def _():
            pltpu.make_async_copy(src3d.at[s, pl.ds(0, IDXCH)],
                                  src_c.at[0], sem_i).wait()
            pltpu.make_async_copy(dst3d.at[s, pl.ds(0, IDXCH)],
                                  dst_c.at[0], sem_i).wait()

        def step(kk):
            buf = bufs[kk]
            pltpu.make_async_copy(xws_hbm.at[src_c.at[p, r8]],
                                  buf, sem_g).wait()
            pltpu.async_copy(buf, acc_sp.at[dst_c.at[p, r8]], sem_s,
                             add=True)

            @pl.when(j < NBLK - 2)
            def _():
                p2 = lax.rem(lax.div(j + 2, IDXCH), 2)
                r2 = lax.rem(j + 2, IDXCH)
                pltpu.async_copy(xws_hbm.at[src_c.at[p2, r2]],
                                 bufs[(kk + 2) % 3], sem_g)

        r3 = lax.rem(j, 3)

        @pl.when(r3 == 0)
        def _():
            step(0)

        @pl.when(r3 == 1)
        def _():
            step(1)

        @pl.when(r3 == 2)
        def _():
            step(2)

        @pl.when(in_t)
        def _():
            pltpu.make_async_copy(dis_sp.at[dst_c.at[0, 0]],
                                  vbuf.at[pl.ds(0, BLK)], sem_t).wait()
            pltpu.async_copy(vbuf.at[pl.ds(0, BLK)],
                             t_sp.at[src_c.at[p, r8]], sem_ts, add=True)
        return 0
    lax.fori_loop(0, NBLK, body, 0)
    pltpu.make_async_copy(bufs[0], acc_sp.at[dst_c.at[0, 0]], sem_s).wait()

    @pl.when(t_end == NBLK)
    def _():
        pltpu.make_async_copy(vbuf.at[pl.ds(0, BLK)],
                              t_sp.at[src_c.at[0, 0]], sem_ts).wait()


def _k4_readout(acc_sp, t_sp, acc_out, t_out, s):
    rsl = pl.ds(s * ROWS_PER_TILE, ROWS_PER_TILE)
    pltpu.sync_copy(acc_sp.at[rsl], acc_out.at[rsl])
    pltpu.sync_copy(t_sp.at[rsl], t_out.at[rsl])


def _sc_scatter_kernel(src3d, dst3d, dis_hbm, xws0, xws1,
                       acc0_out, acc1_out, t0_out, t1_out,
                       src_c, dst_c, rowbufA, rowbufB, rowbufC,
                       vbuf, z_v, acc_sp, t_sp, dis_sp,
                       sem_i, sem_g, sem_s, sem_t, sem_ts):
    c = lax.axis_index("c")
    s = lax.axis_index("s")
    bufs = (rowbufA, rowbufB, rowbufC)

    _zero_rows(rowbufA, BLK)
    _zero_1d(z_v, ROWS_PER_TILE)
    base = s * ROWS_PER_TILE
    for k in range(ROWS_PER_TILE // 80):
        pltpu.sync_copy(rowbufA.at[pl.ds(0, 80)],
                        acc_sp.at[pl.ds(base + k * 80, 80)])
    pltpu.sync_copy(z_v, t_sp.at[pl.ds(base, ROWS_PER_TILE)])
    pltpu.sync_copy(dis_hbm.at[pl.ds(base, ROWS_PER_TILE)],
                    dis_sp.at[pl.ds(base, ROWS_PER_TILE)])
    plsc.subcore_barrier()

    @pl.when(c == 0)
    def _():
        _k4_main_loop(xws0, src3d, dst3d, s, c, src_c, dst_c,
                      bufs, vbuf, acc_sp, t_sp, dis_sp,
                      sem_i, sem_g, sem_s, sem_t, sem_ts)

    @pl.when(c == 1)
    def _():
        _k4_main_loop(xws1, src3d, dst3d, s, c, src_c, dst_c,
                      bufs, vbuf, acc_sp, t_sp, dis_sp,
                      sem_i, sem_g, sem_s, sem_t, sem_ts)

    plsc.subcore_barrier()

    @pl.when(c == 0)
    def _():
        _k4_readout(acc_sp, t_sp, acc0_out, t0_out, s)

    @pl.when(c == 1)
    def _():
        _k4_readout(acc_sp, t_sp, acc1_out, t1_out, s)


@functools.partial(
    pl.kernel,
    mesh=_mesh,
    out_type=[
        jax.ShapeDtypeStruct((NPAD, HALF), jnp.float32),
        jax.ShapeDtypeStruct((NPAD, HALF), jnp.float32),
        jax.ShapeDtypeStruct((NPAD,), jnp.float32),
        jax.ShapeDtypeStruct((NPAD,), jnp.float32),
    ],
    scratch_types=[
        pltpu.VMEM((2, IDXCH, BLK), jnp.int32),
        pltpu.VMEM((2, IDXCH, BLK), jnp.int32),
        pltpu.VMEM((BLK, HALF), jnp.float32),
        pltpu.VMEM((BLK, HALF), jnp.float32),
        pltpu.VMEM((BLK, HALF), jnp.float32),
        pltpu.VMEM((128,), jnp.float32),
        pltpu.VMEM((ROWS_PER_TILE,), jnp.float32),
        pltpu.VMEM_SHARED((NPAD, HALF), jnp.float32),
        pltpu.VMEM_SHARED((NPAD,), jnp.float32),
        pltpu.VMEM_SHARED((NPAD,), jnp.float32),
        pltpu.SemaphoreType.DMA,
        pltpu.SemaphoreType.DMA,
        pltpu.SemaphoreType.DMA,
        pltpu.SemaphoreType.DMA,
        pltpu.SemaphoreType.DMA,
    ],
)
def _sc_scatter(src3d, dst3d, dis_hbm, xws0, xws1,
                acc0_out, acc1_out, t0_out, t1_out,
                src_c, dst_c, rowbufA, rowbufB, rowbufC,
                vbuf, z_v, acc_sp, t_sp, dis_sp,
                sem_i, sem_g, sem_s, sem_t, sem_ts):
    _sc_scatter_kernel(src3d, dst3d, dis_hbm, xws0, xws1,
                       acc0_out, acc1_out, t0_out, t1_out,
                       src_c, dst_c, rowbufA, rowbufB, rowbufC,
                       vbuf, z_v, acc_sp, t_sp, dis_sp,
                       sem_i, sem_g, sem_s, sem_t, sem_ts)


def _tc_final_kernel(acc0_ref, acc1_ref, xws0_ref, xws1_ref, dis_ref,
                     t0_ref, t1_ref, b1_ref, w2_ref, b2_ref,
                     out_ref, vacc):
    i = pl.program_id(0)
    R = acc0_ref.shape[0]
    dis = dis_ref[...]
    dis2 = jnp.reshape(dis, (R, 1))
    b1 = jnp.reshape(b1_ref[...], (1, HID_CH))
    h0 = jnp.maximum(dis2 * (acc0_ref[...] + xws0_ref[...]) + b1[:, :HALF],
                     0.0)
    h1 = jnp.maximum(dis2 * (acc1_ref[...] + xws1_ref[...]) + b1[:, HALF:],
                     0.0)
    cvec = dis * (t0_ref[...] + t1_ref[...] + dis)
    cvec2 = jnp.reshape(cvec, (R, 1))
    rows = lax.broadcasted_iota(jnp.int32, (R, 1), 0) + i * R
    cvec2 = jnp.where(rows < N_NODES, cvec2, 0.0)
    contrib = jnp.concatenate(
        [jnp.sum(cvec2 * h0, axis=0, keepdims=True),
         jnp.sum(cvec2 * h1, axis=0, keepdims=True)], axis=1)

    @pl.when(i == 0)
    def _():
        vacc[...] = contrib

    @pl.when(i > 0)
    def _():
        vacc[...] = vacc[...] + contrib

    @pl.when(i == pl.num_programs(0) - 1)
    def _():
        v = vacc[...]
        o = jnp.dot(v, w2_ref[...], preferred_element_type=jnp.float32)
        out_ref[...] = o * (1.0 / N_NODES) + b2_ref[...]


def _tc_final(acc0, acc1, xws0, xws1, dis, t0, t1, b1, W2, b2):
    R = 512
    grid = (NPAD // R,)
    return pl.pallas_call(
        _tc_final_kernel,
        grid=grid,
        in_specs=[
            pl.BlockSpec((R, HALF), lambda i: (i, 0)),
            pl.BlockSpec((R, HALF), lambda i: (i, 0)),
            pl.BlockSpec((R, HALF), lambda i: (i, 0)),
            pl.BlockSpec((R, HALF), lambda i: (i, 0)),
            pl.BlockSpec((R,), lambda i: (i,)),
            pl.BlockSpec((R,), lambda i: (i,)),
            pl.BlockSpec((R,), lambda i: (i,)),
            pl.BlockSpec((HID_CH,), lambda i: (0,)),
            pl.BlockSpec((HID_CH, OUT_CH), lambda i: (0, 0)),
            pl.BlockSpec((1, OUT_CH), lambda i: (0, 0)),
        ],
        out_specs=pl.BlockSpec((1, OUT_CH), lambda i: (0, 0)),
        out_shape=jax.ShapeDtypeStruct((1, OUT_CH), jnp.float32),
        scratch_shapes=[pltpu.VMEM((1, HID_CH), jnp.float32)],
    )(acc0, acc1, xws0, xws1, dis, t0, t1, b1, W2, b2)


def kernel(x, edge_index, W1, b1, W2, b2):
    src = edge_index[0].astype(jnp.int32)
    dst = edge_index[1].astype(jnp.int32)
    src3d = src.reshape(16, NBLK, BLK)
    dst3d = dst.reshape(16, NBLK, BLK)
    x_pad = jnp.pad(x, ((0, NPAD - N_NODES), (0, 0)))

    deg0, deg1 = _sc_deg(dst3d)
    dis, xws0, xws1 = _tc_scale(x_pad, W1, deg0, deg1)
    acc0, acc1, t0, t1 = _sc_scatter(src3d, dst3d, dis, xws0, xws1)
    out = _tc_final(acc0, acc1, xws0, xws1, dis, t0, t1, b1, W2,
                    b2.reshape(1, OUT_CH))
    return out.reshape(OUT_CH)
